# baseline (speedup 1.0000x reference)
import functools

import jax
import jax.numpy as jnp
from jax import lax
from jax.experimental import pallas as pl
from jax.experimental.pallas import tpu as pltpu
from jax.experimental.pallas import tpu_sc as plsc

N_NODES = 10000
D = 128
DH = D // 2
N_EDGES = 320000

NC = 2
NS = 16
CHUNK = 128
NBUF = 4
PD = 1

E_PER_W = N_EDGES // NS
NT = -(-E_PER_W // CHUNK)
NT = -(-NT // NBUF) * NBUF
E_PER_W_PAD = NT * CHUNK
SEG = 8
NTSEG = NT // SEG
AGG_ROWS = 10240
N_PAD = AGG_ROWS
N_PER_TILE = N_PAD // NS
NSEG = 4
NPT_SEG = N_PER_TILE // NSEG
REM = N_NODES % NPT_SEG

_mesh = plsc.VectorSubcoreMesh(core_axis_name="c", subcore_axis_name="s")


@functools.partial(
    pl.kernel,
    mesh=_mesh,
    compiler_params=pltpu.CompilerParams(use_tc_tiling_on_sc=False),
    out_type=jax.ShapeDtypeStruct((N_NODES, D), jnp.float32),
    scratch_types=[
        pltpu.VMEM((NTSEG, CHUNK), jnp.int32),
        pltpu.VMEM((NTSEG, CHUNK), jnp.int32),
        pltpu.VMEM((CHUNK, DH), jnp.float32),
        pltpu.VMEM((CHUNK, DH), jnp.float32),
        pltpu.VMEM((CHUNK, DH), jnp.float32),
        pltpu.VMEM((CHUNK, DH), jnp.float32),
        pltpu.VMEM((NPT_SEG, DH), jnp.float32),
        pltpu.VMEM((N_PER_TILE,), jnp.float32),
        pltpu.VMEM_SHARED((N_PAD, DH), jnp.float32),
        pltpu.VMEM_SHARED((AGG_ROWS, DH), jnp.float32),
        pltpu.SemaphoreType.DMA,
        pltpu.SemaphoreType.DMA,
        pltpu.SemaphoreType.DMA,
        pltpu.SemaphoreType.DMA,
        pltpu.SemaphoreType.DMA,
        pltpu.SemaphoreType.DMA,
        pltpu.SemaphoreType.DMA,
        pltpu.SemaphoreType.DMA,
    ],
)
def _gcn_sc(node_hbm, norm_hbm, src_hbm, dst_hbm, rst_hbm,
            idx_s, idx_d, r0, r1, r2, r3, nf_loc, norm_loc, h_sp, agg,
            g0, g1, g2, g3, s0, s1, s2, s3):
    cid = lax.axis_index("c")
    sid = lax.axis_index("s")
    rows = (r0, r1, r2, r3)
    gsem = (g0, g1, g2, g3)
    ssem = (s0, s1, s2, s3)
    row0 = sid * N_PER_TILE

    pltpu.sync_copy(norm_hbm.at[sid], norm_loc)
    for ns in range(NSEG):
        r0n = row0 + ns * NPT_SEG
        full = r0n + NPT_SEG <= N_NODES
        ragged = jnp.logical_and(r0n < N_NODES, r0n + NPT_SEG > N_NODES)

        @pl.when(full)
        def _ld_full(r0n=r0n):
            pltpu.sync_copy(
                node_hbm.at[pl.ds(r0n, NPT_SEG), pl.ds(cid * DH, DH)], nf_loc)

        @pl.when(ragged)
        def _ld_part(r0n=r0n):
            pltpu.sync_copy(
                node_hbm.at[pl.ds(r0n, REM), pl.ds(cid * DH, DH)],
                nf_loc.at[pl.ds(0, REM)])

        def _pre(g, _, ns=ns):
            nv = norm_loc[pl.ds(ns * NPT_SEG + g * 16, 16)]
            for l in range(16):
                v = nv[l]
                for j in range(DH // 16):
                    sl = pl.ds(j * 16, 16)
                    nf_loc[g * 16 + l, sl] = nf_loc[g * 16 + l, sl] * v
            return 0

        lax.fori_loop(0, NPT_SEG // 16, _pre, 0)

        @pl.when(full)
        def _st_full(r0n=r0n):
            pltpu.sync_copy(nf_loc, h_sp.at[pl.ds(r0n, NPT_SEG)])

        @pl.when(ragged)
        def _st_part(r0n=r0n):
            pltpu.sync_copy(nf_loc.at[pl.ds(0, REM)],
                            h_sp.at[pl.ds(r0n, REM)])

    zeros16 = jnp.zeros((16,), jnp.float32)

    def _zrow(i, _):
        for j in range(DH // 16):
            r0[i, pl.ds(j * 16, 16)] = zeros16
        return 0

    lax.fori_loop(0, CHUNK, _zrow, 0)
    for j in range(AGG_ROWS // NS // CHUNK):
        pltpu.sync_copy(
            r0, agg.at[pl.ds(sid * (AGG_ROWS // NS) + j * CHUNK, CHUNK)])

    plsc.subcore_barrier()

    for seg in range(SEG):
        pltpu.sync_copy(src_hbm.at[sid, pl.ds(seg * NTSEG, NTSEG)], idx_s)
        pltpu.sync_copy(dst_hbm.at[sid, pl.ds(seg * NTSEG, NTSEG)], idx_d)
        for b in range(PD):
            pltpu.async_copy(h_sp.at[idx_s.at[b]], rows[b], gsem[b])

        def _outer(si, _):
            for b in range(NBUF):
                lt = si * NBUF + b
                b2 = (b + PD) % NBUF

                @pl.when(lt + PD - NBUF >= 0)
                def _wait_scatter():
                    pltpu.make_async_copy(
                        rows[b2], agg.at[idx_d.at[lt + PD - NBUF]],
                        ssem[b2]).wait()

                @pl.when(lt + PD < NTSEG)
                def _prefetch():
                    pltpu.async_copy(h_sp.at[idx_s.at[lt + PD]], rows[b2],
                                     gsem[b2])

                pltpu.make_async_copy(h_sp.at[idx_s.at[lt]], rows[b],
                                      gsem[b]).wait()
                pltpu.async_copy(rows[b], agg.at[idx_d.at[lt]], ssem[b],
                                 add=True)
            return 0

        lax.fori_loop(0, NTSEG // NBUF, _outer, 0)
        for k in range(NBUF - PD):
            c = NTSEG - (NBUF - PD) + k
            pltpu.make_async_copy(rows[c % NBUF], agg.at[idx_d.at[c]],
                                  ssem[c % NBUF]).wait()
    plsc.subcore_barrier()

    for ns in range(NSEG):
        r0n = row0 + ns * NPT_SEG
        full = r0n + NPT_SEG <= N_NODES
        ragged = jnp.logical_and(r0n < N_NODES, r0n + NPT_SEG > N_NODES)
        pltpu.sync_copy(agg.at[pl.ds(r0n, NPT_SEG)], nf_loc)

        def _post(g, _, ns=ns):
            nv = norm_loc[pl.ds(ns * NPT_SEG + g * 16, 16)]
            for l in range(16):
                v = nv[l]
                for j in range(DH // 16):
                    sl = pl.ds(j * 16, 16)
                    nf_loc[g * 16 + l, sl] = nf_loc[g * 16 + l, sl] * v
            return 0

        lax.fori_loop(0, NPT_SEG // 16, _post, 0)

        @pl.when(full)
        def _st_full(r0n=r0n):
            pltpu.sync_copy(
                nf_loc, rst_hbm.at[pl.ds(r0n, NPT_SEG), pl.ds(cid * DH, DH)])

        @pl.when(ragged)
        def _st_part(r0n=r0n):
            pltpu.sync_copy(
                nf_loc.at[pl.ds(0, REM)],
                rst_hbm.at[pl.ds(r0n, REM), pl.ds(cid * DH, DH)])


def kernel(node_f, norm, edge_index):
    src = edge_index[0].astype(jnp.int32)
    dst = edge_index[1].astype(jnp.int32)

    pad = E_PER_W_PAD - E_PER_W
    src_p = jnp.pad(src.reshape(NS, E_PER_W), ((0, 0), (0, pad))
                    ).reshape(NS, NT, CHUNK)
    dst_p = jnp.pad(dst.reshape(NS, E_PER_W), ((0, 0), (0, pad)),
                    constant_values=N_NODES).reshape(NS, NT, CHUNK)
    norm_r = jnp.pad(norm.reshape(-1), (0, N_PAD - N_NODES)).reshape(NS, N_PER_TILE)

    return _gcn_sc(node_f, norm_r, src_p, dst_p)


# speedup vs baseline: 1.0007x; 1.0007x over previous
import functools

import jax
import jax.numpy as jnp
from jax import lax
from jax.experimental import pallas as pl
from jax.experimental.pallas import tpu as pltpu
from jax.experimental.pallas import tpu_sc as plsc

N_NODES = 10000
D = 128
DH = D // 2
N_EDGES = 320000

NC = 2
NS = 16
CHUNK = 128
NBUF = 4
PD = 2

E_PER_W = N_EDGES // NS
NT = -(-E_PER_W // CHUNK)
NT = -(-NT // NBUF) * NBUF
E_PER_W_PAD = NT * CHUNK
SEG = 8
NTSEG = NT // SEG
AGG_ROWS = 10240
N_PAD = AGG_ROWS
N_PER_TILE = N_PAD // NS
NSEG = 4
NPT_SEG = N_PER_TILE // NSEG
REM = N_NODES % NPT_SEG

_mesh = plsc.VectorSubcoreMesh(core_axis_name="c", subcore_axis_name="s")


@functools.partial(
    pl.kernel,
    mesh=_mesh,
    compiler_params=pltpu.CompilerParams(use_tc_tiling_on_sc=False),
    out_type=jax.ShapeDtypeStruct((N_NODES, D), jnp.float32),
    scratch_types=[
        pltpu.VMEM((NTSEG, CHUNK), jnp.int32),
        pltpu.VMEM((NTSEG, CHUNK), jnp.int32),
        pltpu.VMEM((CHUNK, DH), jnp.float32),
        pltpu.VMEM((CHUNK, DH), jnp.float32),
        pltpu.VMEM((CHUNK, DH), jnp.float32),
        pltpu.VMEM((CHUNK, DH), jnp.float32),
        pltpu.VMEM((NPT_SEG, DH), jnp.float32),
        pltpu.VMEM((N_PER_TILE,), jnp.float32),
        pltpu.VMEM_SHARED((N_PAD, DH), jnp.float32),
        pltpu.VMEM_SHARED((AGG_ROWS, DH), jnp.float32),
        pltpu.SemaphoreType.DMA,
        pltpu.SemaphoreType.DMA,
        pltpu.SemaphoreType.DMA,
        pltpu.SemaphoreType.DMA,
        pltpu.SemaphoreType.DMA,
        pltpu.SemaphoreType.DMA,
        pltpu.SemaphoreType.DMA,
        pltpu.SemaphoreType.DMA,
    ],
)
def _gcn_sc(node_hbm, norm_hbm, src_hbm, dst_hbm, rst_hbm,
            idx_s, idx_d, r0, r1, r2, r3, nf_loc, norm_loc, h_sp, agg,
            g0, g1, g2, g3, s0, s1, s2, s3):
    cid = lax.axis_index("c")
    sid = lax.axis_index("s")
    rows = (r0, r1, r2, r3)
    gsem = (g0, g1, g2, g3)
    ssem = (s0, s1, s2, s3)
    row0 = sid * N_PER_TILE

    pltpu.sync_copy(norm_hbm.at[sid], norm_loc)
    for ns in range(NSEG):
        r0n = row0 + ns * NPT_SEG
        full = r0n + NPT_SEG <= N_NODES
        ragged = jnp.logical_and(r0n < N_NODES, r0n + NPT_SEG > N_NODES)

        @pl.when(full)
        def _ld_full(r0n=r0n):
            pltpu.sync_copy(
                node_hbm.at[pl.ds(r0n, NPT_SEG), pl.ds(cid * DH, DH)], nf_loc)

        @pl.when(ragged)
        def _ld_part(r0n=r0n):
            pltpu.sync_copy(
                node_hbm.at[pl.ds(r0n, REM), pl.ds(cid * DH, DH)],
                nf_loc.at[pl.ds(0, REM)])

        def _pre(g, _, ns=ns):
            nv = norm_loc[pl.ds(ns * NPT_SEG + g * 16, 16)]
            for l in range(16):
                v = nv[l]
                for j in range(DH // 16):
                    sl = pl.ds(j * 16, 16)
                    nf_loc[g * 16 + l, sl] = nf_loc[g * 16 + l, sl] * v
            return 0

        lax.fori_loop(0, NPT_SEG // 16, _pre, 0)

        @pl.when(full)
        def _st_full(r0n=r0n):
            pltpu.sync_copy(nf_loc, h_sp.at[pl.ds(r0n, NPT_SEG)])

        @pl.when(ragged)
        def _st_part(r0n=r0n):
            pltpu.sync_copy(nf_loc.at[pl.ds(0, REM)],
                            h_sp.at[pl.ds(r0n, REM)])

    zeros16 = jnp.zeros((16,), jnp.float32)

    def _zrow(i, _):
        for j in range(DH // 16):
            r0[i, pl.ds(j * 16, 16)] = zeros16
        return 0

    lax.fori_loop(0, CHUNK, _zrow, 0)
    for j in range(AGG_ROWS // NS // CHUNK):
        pltpu.sync_copy(
            r0, agg.at[pl.ds(sid * (AGG_ROWS // NS) + j * CHUNK, CHUNK)])

    plsc.subcore_barrier()

    for seg in range(SEG):
        pltpu.sync_copy(src_hbm.at[sid, pl.ds(seg * NTSEG, NTSEG)], idx_s)
        pltpu.sync_copy(dst_hbm.at[sid, pl.ds(seg * NTSEG, NTSEG)], idx_d)
        for b in range(PD):
            pltpu.async_copy(h_sp.at[idx_s.at[b]], rows[b], gsem[b])

        def _outer(si, _):
            for b in range(NBUF):
                lt = si * NBUF + b
                b2 = (b + PD) % NBUF

                @pl.when(lt + PD - NBUF >= 0)
                def _wait_scatter():
                    pltpu.make_async_copy(
                        rows[b2], agg.at[idx_d.at[lt + PD - NBUF]],
                        ssem[b2]).wait()

                @pl.when(lt + PD < NTSEG)
                def _prefetch():
                    pltpu.async_copy(h_sp.at[idx_s.at[lt + PD]], rows[b2],
                                     gsem[b2])

                pltpu.make_async_copy(h_sp.at[idx_s.at[lt]], rows[b],
                                      gsem[b]).wait()
                pltpu.async_copy(rows[b], agg.at[idx_d.at[lt]], ssem[b],
                                 add=True)
            return 0

        lax.fori_loop(0, NTSEG // NBUF, _outer, 0)
        for k in range(NBUF - PD):
            c = NTSEG - (NBUF - PD) + k
            pltpu.make_async_copy(rows[c % NBUF], agg.at[idx_d.at[c]],
                                  ssem[c % NBUF]).wait()
    plsc.subcore_barrier()

    for ns in range(NSEG):
        r0n = row0 + ns * NPT_SEG
        full = r0n + NPT_SEG <= N_NODES
        ragged = jnp.logical_and(r0n < N_NODES, r0n + NPT_SEG > N_NODES)
        pltpu.sync_copy(agg.at[pl.ds(r0n, NPT_SEG)], nf_loc)

        def _post(g, _, ns=ns):
            nv = norm_loc[pl.ds(ns * NPT_SEG + g * 16, 16)]
            for l in range(16):
                v = nv[l]
                for j in range(DH // 16):
                    sl = pl.ds(j * 16, 16)
                    nf_loc[g * 16 + l, sl] = nf_loc[g * 16 + l, sl] * v
            return 0

        lax.fori_loop(0, NPT_SEG // 16, _post, 0)

        @pl.when(full)
        def _st_full(r0n=r0n):
            pltpu.sync_copy(
                nf_loc, rst_hbm.at[pl.ds(r0n, NPT_SEG), pl.ds(cid * DH, DH)])

        @pl.when(ragged)
        def _st_part(r0n=r0n):
            pltpu.sync_copy(
                nf_loc.at[pl.ds(0, REM)],
                rst_hbm.at[pl.ds(r0n, REM), pl.ds(cid * DH, DH)])


def kernel(node_f, norm, edge_index):
    src = edge_index[0].astype(jnp.int32)
    dst = edge_index[1].astype(jnp.int32)

    pad = E_PER_W_PAD - E_PER_W
    src_p = jnp.pad(src.reshape(NS, E_PER_W), ((0, 0), (0, pad))
                    ).reshape(NS, NT, CHUNK)
    dst_p = jnp.pad(dst.reshape(NS, E_PER_W), ((0, 0), (0, pad)),
                    constant_values=N_NODES).reshape(NS, NT, CHUNK)
    norm_r = jnp.pad(norm.reshape(-1), (0, N_PAD - N_NODES)).reshape(NS, N_PER_TILE)

    return _gcn_sc(node_f, norm_r, src_p, dst_p)
